# baseline (speedup 1.0000x reference)
import functools

import jax
import jax.numpy as jnp
from jax import lax
from jax.experimental import pallas as pl
from jax.experimental.pallas import tpu as pltpu

N = 10000
D = 512
K = 6
KPAD = 8
NEG_BIG = -3e38

_MM_ROWS = 2000
_Q = 200
_SC_ROWS = 1000


def _matmul_body(x_ref, w_ref, a_ref, b_ref):
    w1 = w_ref[:, :D]
    w2 = w_ref[:, D:]
    x = x_ref[...]
    dn = (((1,), (1,)), ((), ()))
    a_ref[...] = lax.dot_general(x, w1 - w2, dn, preferred_element_type=jnp.float32)
    b_ref[...] = lax.dot_general(x, w2, dn, preferred_element_type=jnp.float32)


def _matmul_call(x, w):
    grid = (N // _MM_ROWS,)
    return pl.pallas_call(
        _matmul_body,
        grid=grid,
        in_specs=[
            pl.BlockSpec((_MM_ROWS, D), lambda i: (i, 0)),
            pl.BlockSpec((D, 2 * D), lambda i: (0, 0)),
        ],
        out_specs=[
            pl.BlockSpec((_MM_ROWS, D), lambda i: (i, 0)),
            pl.BlockSpec((_MM_ROWS, D), lambda i: (i, 0)),
        ],
        out_shape=[
            jax.ShapeDtypeStruct((N, D), jnp.float32),
            jax.ShapeDtypeStruct((N, D), jnp.float32),
        ],
    )(x, w)


def _knn_body(q_ref, pt_ref, nbr_ref):
    blk = pl.program_id(0)
    q = q_ref[...]
    qx = q[:, 0:1]
    qy = q[:, 1:2]
    qz = q[:, 2:3]
    px = pt_ref[0:1, :]
    py = pt_ref[1:2, :]
    pz = pt_ref[2:3, :]
    qsq = qx * qx + qy * qy + qz * qz
    csq = px * px + py * py + pz * pz
    dot = lax.dot_general(
        q.astype(jnp.bfloat16), pt_ref[...].astype(jnp.bfloat16),
        (((1,), (0,)), ((), ())), preferred_element_type=jnp.float32)
    d = qsq + csq - 2.0 * dot
    lane = lax.broadcasted_iota(jnp.int32, (_Q, N), 1)
    row = lax.broadcasted_iota(jnp.int32, (_Q, N), 0)
    gid = row + blk * _Q
    d = jnp.where(lane == gid, jnp.inf, d)
    for j in range(K):
        m = jnp.min(d, axis=1, keepdims=True)
        sel = jnp.where(d == m, lane, jnp.int32(2**30))
        idx = jnp.min(sel, axis=1, keepdims=True)
        nbr_ref[:, j:j + 1] = idx
        d = jnp.where(lane == idx, jnp.inf, d)
    nbr_ref[:, K:K + 1] = jnp.zeros((_Q, 1), jnp.int32)
    nbr_ref[:, K + 1:K + 2] = jnp.zeros((_Q, 1), jnp.int32)


def _knn_call(pos_q, pos_t):
    grid = (N // _Q,)
    return pl.pallas_call(
        _knn_body,
        grid=grid,
        in_specs=[
            pl.BlockSpec((_Q, 8), lambda i: (i, 0)),
            pl.BlockSpec((8, N), lambda i: (0, 0)),
        ],
        out_specs=pl.BlockSpec((_Q, KPAD), lambda i: (i, 0)),
        out_shape=jax.ShapeDtypeStruct((N, KPAD), jnp.int32),
    )(pos_q, pos_t)


def _scatter_body(nbr_ref, b_ref, m_ref):
    step = pl.program_id(0)

    @pl.when(step == 0)
    def _init():
        def ib(k, _):
            m_ref[pl.ds(k * 80, 80)] = jnp.full((80, 4, 128), NEG_BIG,
                                                jnp.float32)
            return 0
        lax.fori_loop(0, N // 80, ib, 0)

    def body(i, _):
        brow = b_ref[pl.ds(i, 1)]
        for j in range(K):
            c = nbr_ref[i, j]
            cur = m_ref[pl.ds(c, 1)]
            m_ref[pl.ds(c, 1)] = jnp.maximum(cur, brow)
        return 0

    lax.fori_loop(0, _SC_ROWS, body, 0)


def _scatter_call(nbr, b_mat):
    grid = (N // _SC_ROWS,)
    return pl.pallas_call(
        _scatter_body,
        grid=grid,
        in_specs=[
            pl.BlockSpec((_SC_ROWS, KPAD), lambda i: (i, 0), memory_space=pltpu.SMEM),
            pl.BlockSpec((_SC_ROWS, 4, 128), lambda i: (i, 0, 0)),
        ],
        out_specs=pl.BlockSpec((N, 4, 128), lambda i: (0, 0, 0)),
        out_shape=jax.ShapeDtypeStruct((N, 4, 128), jnp.float32),
    )(nbr, jnp.reshape(b_mat, (N, 4, 128)))


def _final_body(m_ref, a_ref, bias_ref, o_ref):
    o_ref[...] = jnp.maximum(m_ref[...] + a_ref[...] + bias_ref[...], 0.0)


def _final_call(m, a, bias):
    grid = (N // _SC_ROWS,)
    return pl.pallas_call(
        _final_body,
        grid=grid,
        in_specs=[
            pl.BlockSpec((_SC_ROWS, D), lambda i: (i, 0)),
            pl.BlockSpec((_SC_ROWS, D), lambda i: (i, 0)),
            pl.BlockSpec((1, D), lambda i: (0, 0)),
        ],
        out_specs=pl.BlockSpec((_SC_ROWS, D), lambda i: (i, 0)),
        out_shape=jax.ShapeDtypeStruct((N, D), jnp.float32),
    )(m, a, bias)


@jax.jit
def kernel(x, pos, W, b):
    pos_t = jnp.zeros((8, N), jnp.float32).at[:3, :].set(pos.T)
    pos_q = jnp.zeros((N, 8), jnp.float32).at[:, :3].set(pos)
    a_mat, b_mat = _matmul_call(x, W)
    nbr = _knn_call(pos_q, pos_t)
    m = jnp.reshape(_scatter_call(nbr, b_mat), (N, D))
    return _final_call(m, a_mat, jnp.reshape(b, (1, D)))


# speedup vs baseline: 1.0872x; 1.0872x over previous
import functools

import jax
import jax.numpy as jnp
from jax import lax
from jax.experimental import pallas as pl
from jax.experimental.pallas import tpu as pltpu

N = 10000
D = 512
K = 6
KPAD = 8
NEG_BIG = -3e38

_MM_ROWS = 2000
_Q = 400
_SC_ROWS = 1000


def _matmul_body(x_ref, w_ref, a_ref, b_ref):
    w1 = w_ref[:, :D]
    w2 = w_ref[:, D:]
    x = x_ref[...]
    dn = (((1,), (1,)), ((), ()))
    a_ref[...] = lax.dot_general(x, w1 - w2, dn, preferred_element_type=jnp.float32)
    b_ref[...] = lax.dot_general(x, w2, dn, preferred_element_type=jnp.float32)


def _matmul_call(x, w):
    grid = (N // _MM_ROWS,)
    return pl.pallas_call(
        _matmul_body,
        grid=grid,
        in_specs=[
            pl.BlockSpec((_MM_ROWS, D), lambda i: (i, 0)),
            pl.BlockSpec((D, 2 * D), lambda i: (0, 0)),
        ],
        out_specs=[
            pl.BlockSpec((_MM_ROWS, D), lambda i: (i, 0)),
            pl.BlockSpec((_MM_ROWS, D), lambda i: (i, 0)),
        ],
        out_shape=[
            jax.ShapeDtypeStruct((N, D), jnp.float32),
            jax.ShapeDtypeStruct((N, D), jnp.float32),
        ],
    )(x, w)


def _knn_body(q_ref, pt_ref, nbr_ref):
    blk = pl.program_id(0)
    q = q_ref[...]
    qx = q[:, 0:1]
    qy = q[:, 1:2]
    qz = q[:, 2:3]
    px = pt_ref[0:1, :]
    py = pt_ref[1:2, :]
    pz = pt_ref[2:3, :]
    qsq = qx * qx + qy * qy + qz * qz
    csq = px * px + py * py + pz * pz
    dot = lax.dot_general(
        q.astype(jnp.bfloat16), pt_ref[...].astype(jnp.bfloat16),
        (((1,), (0,)), ((), ())), preferred_element_type=jnp.float32)
    d = qsq + csq - 2.0 * dot
    lane = lax.broadcasted_iota(jnp.int32, (_Q, N), 1)
    row = lax.broadcasted_iota(jnp.int32, (_Q, N), 0)
    gid = row + blk * _Q
    d = jnp.where(lane == gid, jnp.inf, d)
    for j in range(K):
        m = jnp.min(d, axis=1, keepdims=True)
        sel = jnp.where(d == m, lane, jnp.int32(2**30))
        idx = jnp.min(sel, axis=1, keepdims=True)
        nbr_ref[:, j:j + 1] = idx
        if j < K - 1:
            d = jnp.where(lane == idx, jnp.inf, d)
    nbr_ref[:, K:K + 1] = jnp.zeros((_Q, 1), jnp.int32)
    nbr_ref[:, K + 1:K + 2] = jnp.zeros((_Q, 1), jnp.int32)


def _knn_call(pos_q, pos_t):
    grid = (N // _Q,)
    return pl.pallas_call(
        _knn_body,
        grid=grid,
        in_specs=[
            pl.BlockSpec((_Q, 8), lambda i: (i, 0)),
            pl.BlockSpec((8, N), lambda i: (0, 0)),
        ],
        out_specs=pl.BlockSpec((_Q, KPAD), lambda i: (i, 0)),
        out_shape=jax.ShapeDtypeStruct((N, KPAD), jnp.int32),
    )(pos_q, pos_t)


def _scatter_body(nbr_ref, b_ref, m_ref):
    step = pl.program_id(0)

    @pl.when(step == 0)
    def _init():
        def ib(k, _):
            m_ref[pl.ds(k * 80, 80), :] = jnp.full((80, D), NEG_BIG, jnp.float32)
            return 0
        lax.fori_loop(0, N // 80, ib, 0)

    def body(i, _):
        brow = b_ref[pl.ds(i, 1), :]
        for j in range(K):
            c = nbr_ref[i, j]
            cur = m_ref[pl.ds(c, 1), :]
            m_ref[pl.ds(c, 1), :] = jnp.maximum(cur, brow)
        return 0

    lax.fori_loop(0, _SC_ROWS, body, 0)


def _scatter_call(nbr, b_mat):
    grid = (N // _SC_ROWS,)
    return pl.pallas_call(
        _scatter_body,
        grid=grid,
        in_specs=[
            pl.BlockSpec((_SC_ROWS, KPAD), lambda i: (i, 0), memory_space=pltpu.SMEM),
            pl.BlockSpec((_SC_ROWS, D), lambda i: (i, 0)),
        ],
        out_specs=pl.BlockSpec((N, D), lambda i: (0, 0)),
        out_shape=jax.ShapeDtypeStruct((N, D), jnp.float32),
    )(nbr, b_mat)


def _final_body(m_ref, a_ref, bias_ref, o_ref):
    o_ref[...] = jnp.maximum(m_ref[...] + a_ref[...] + bias_ref[...], 0.0)


def _final_call(m, a, bias):
    grid = (N // _SC_ROWS,)
    return pl.pallas_call(
        _final_body,
        grid=grid,
        in_specs=[
            pl.BlockSpec((_SC_ROWS, D), lambda i: (i, 0)),
            pl.BlockSpec((_SC_ROWS, D), lambda i: (i, 0)),
            pl.BlockSpec((1, D), lambda i: (0, 0)),
        ],
        out_specs=pl.BlockSpec((_SC_ROWS, D), lambda i: (i, 0)),
        out_shape=jax.ShapeDtypeStruct((N, D), jnp.float32),
    )(m, a, bias)


@jax.jit
def kernel(x, pos, W, b):
    pos_t = jnp.zeros((8, N), jnp.float32).at[:3, :].set(pos.T)
    pos_q = jnp.zeros((N, 8), jnp.float32).at[:, :3].set(pos)
    a_mat, b_mat = _matmul_call(x, W)
    nbr = _knn_call(pos_q, pos_t)
    m = _scatter_call(nbr, b_mat)
    return _final_call(m, a_mat, jnp.reshape(b, (1, D)))
